# baseline (speedup 1.0000x reference)
import jax
import jax.numpy as jnp
from jax import lax
from jax.experimental import pallas as pl
from jax.experimental.pallas import tpu as pltpu
from jax.experimental.pallas import tpu_sc as plsc

D = 512
LANES = 16
KD = D // LANES
EPS = 1e-5
NW = 32
CH = 32
NBUF = 4
N_CHUNKS = 32
N_PLAIN = 16
TC_TS = 512


def _allreduce_sum(v):
    idx = lax.iota(jnp.int32, LANES)
    for sh in (8, 4, 2, 1):
        v = v + v[(idx + sh) & (LANES - 1)]
    return v


def _rsqrt_newton(v):
    bits = lax.bitcast_convert_type(v, jnp.int32)
    y = lax.bitcast_convert_type(jnp.int32(0x5F3759DF) - (bits >> 1), jnp.float32)
    for _ in range(4):
        y = y * (1.5 - 0.5 * v * y * y)
    return y


def _make_sc_stage(n_tok):
    tok_per_w = n_tok // NW
    assert tok_per_w == N_CHUNKS * CH
    mesh = plsc.VectorSubcoreMesh(core_axis_name="c", subcore_axis_name="s")

    def body(x_hbm, table_hbm, pos_hbm, mod_hbm, g_hbm, b_hbm, out_hbm,
             idx_v, r0, r1, r2, r3, p0, p1, mod_v, g_v, b_v,
             mnb, ivb,
             rg0, rg1, rg2, rg3, ro0, ro1, ro2, ro3, pg0, pg1):
        wid = lax.axis_index("s") * 2 + lax.axis_index("c")
        base = wid * tok_per_w
        pos_base = base % 8192

        rbufs = (r0, r1, r2, r3)
        rgs = (rg0, rg1, rg2, rg3)
        ros = (ro0, ro1, ro2, ro3)
        pbufs = (p0, p1)
        pgs = (pg0, pg1)

        pltpu.sync_copy(x_hbm.at[wid], idx_v)
        pltpu.sync_copy(mod_hbm.at[0], mod_v)
        pltpu.sync_copy(g_hbm, g_v)
        pltpu.sync_copy(b_hbm, b_v)

        def gather(cc, rb):
            return pltpu.async_copy(table_hbm.at[idx_v.at[cc]],
                                    rbufs[rb], rgs[rb])

        def put(cc, rb):
            return pltpu.async_copy(rbufs[rb],
                                    out_hbm.at[pl.ds(base + cc * CH, CH)],
                                    ros[rb])

        def posget(cc, pb):
            return pltpu.async_copy(
                pos_hbm.at[pl.ds(pos_base + cc * CH, CH)],
                pbufs[pb], pgs[pb])

        def fused_ln(rb, pb):
            rbuf = rbufs[rb]
            pbuf = pbufs[pb]
            zeros = jnp.zeros((LANES,), jnp.float32)
            for jb in range(CH // 8):
                @plsc.parallel_loop(0, KD, unroll=2, carry=(zeros,) * 16)
                def res(k, carry):
                    accs = list(carry[:8])
                    acc2s = list(carry[8:])
                    sl = pl.ds(k * LANES, LANES)
                    m = mod_v[sl]
                    for j in range(8):
                        t = jb * 8 + j
                        h = rbuf[t, sl] + pbuf[t, sl] + m
                        rbuf[t, sl] = h
                        accs[j] = accs[j] + h
                        acc2s[j] = acc2s[j] + h * h
                    return tuple(accs) + tuple(acc2s)

                for j in range(8):
                    mean = _allreduce_sum(res[j]) * (1.0 / D)
                    var = (_allreduce_sum(res[8 + j]) * (1.0 / D)
                           - mean * mean)
                    mnb[jb * 8 + j] = mean
                    ivb[jb * 8 + j] = _rsqrt_newton(var + EPS)
            for jb in range(CH // 8):
                mjs = []
                ijs = []
                for j in range(8):
                    t = jb * 8 + j
                    mjs.append(mnb[t])
                    ijs.append(ivb[t])

                @plsc.parallel_loop(0, KD, unroll=2)
                def _(k):
                    sl = pl.ds(k * LANES, LANES)
                    gk = g_v[sl]
                    bk = b_v[sl]
                    for j in range(8):
                        t = jb * 8 + j
                        h = rbuf[t, sl]
                        rbuf[t, sl] = (h - mjs[j]) * ijs[j] * gk + bk

        gather(0, 0)
        gather(1, 1)

        def outer(c4, _):
            for b in range(NBUF):
                cc = c4 * NBUF + b

                pltpu.make_async_copy(table_hbm.at[idx_v.at[cc]],
                                      rbufs[b], rgs[b]).wait()
                is_fused = cc >= N_PLAIN

                @pl.when(is_fused)
                def _():
                    pltpu.make_async_copy(
                        pos_hbm.at[pl.ds(pos_base + cc * CH, CH)],
                        pbufs[b % 2], pgs[b % 2]).wait()
                    fused_ln(b, b % 2)

                put(cc, b)

                nb = (b + 2) % NBUF

                @pl.when(jnp.logical_and(cc + 2 < N_CHUNKS, cc >= 2))
                def _():
                    pltpu.make_async_copy(
                        rbufs[nb],
                        out_hbm.at[pl.ds(base + (cc - 2) * CH, CH)],
                        ros[nb]).wait()

                @pl.when(cc + 2 < N_CHUNKS)
                def _():
                    gather(cc + 2, nb)

                @pl.when(jnp.logical_and(cc + 2 >= N_PLAIN,
                                         cc + 2 < N_CHUNKS))
                def _():
                    posget(cc + 2, b % 2)

            return 0

        lax.fori_loop(0, N_CHUNKS // NBUF, outer, 0)

        for cc in range(N_CHUNKS - NBUF, N_CHUNKS):
            b = cc % NBUF
            pltpu.make_async_copy(rbufs[b],
                                  out_hbm.at[pl.ds(base + cc * CH, CH)],
                                  ros[b]).wait()

    return pl.kernel(
        body,
        out_type=jax.ShapeDtypeStruct((n_tok, D), jnp.float32),
        mesh=mesh,
        scratch_types=[
            pltpu.VMEM((N_CHUNKS, CH), jnp.int32),
            pltpu.VMEM((CH, D), jnp.float32),
            pltpu.VMEM((CH, D), jnp.float32),
            pltpu.VMEM((CH, D), jnp.float32),
            pltpu.VMEM((CH, D), jnp.float32),
            pltpu.VMEM((CH, D), jnp.float32),
            pltpu.VMEM((CH, D), jnp.float32),
            pltpu.VMEM((D,), jnp.float32),
            pltpu.VMEM((D,), jnp.float32),
            pltpu.VMEM((D,), jnp.float32),
            pltpu.VMEM((CH, LANES), jnp.float32),
            pltpu.VMEM((CH, LANES), jnp.float32),
        ] + [pltpu.SemaphoreType.DMA] * 10,
    )


def _tc_ln_body(rows_ref, pos_ref, mod_ref, g_ref, b_ref, o_ref):
    h = rows_ref[...] + pos_ref[...][None] + mod_ref[...][None]
    mean = jnp.mean(h, axis=-1, keepdims=True)
    meansq = jnp.mean(h * h, axis=-1, keepdims=True)
    var = meansq - mean * mean
    scale = lax.rsqrt(var + EPS) * g_ref[...][None]
    shift = b_ref[...][None] - mean * scale
    o_ref[...] = h * scale + shift


def _tc_ln_plain_half(rows3d, pos2d, mod_row, g2d, b2d, bsz, seq):
    n_ranges = seq // 1024
    return pl.pallas_call(
        _tc_ln_body,
        grid=(n_ranges,),
        in_specs=[
            pl.BlockSpec((bsz, TC_TS, D), lambda j: (0, 2 * j, 0)),
            pl.BlockSpec((TC_TS, D), lambda j: (2 * j, 0)),
            pl.BlockSpec((1, D), lambda j: (0, 0)),
            pl.BlockSpec((1, D), lambda j: (0, 0)),
            pl.BlockSpec((1, D), lambda j: (0, 0)),
        ],
        out_specs=pl.BlockSpec((bsz, TC_TS, D), lambda j: (0, 2 * j, 0)),
        out_shape=jax.ShapeDtypeStruct((bsz, seq, D), jnp.float32),
        input_output_aliases={0: 0},
    )(rows3d, pos2d, mod_row, g2d, b2d)


def kernel(x, token_table, pos_emb, mod_table, gamma, beta):
    bsz, seq = x.shape
    n_tok = bsz * seq
    x_arr = x.astype(jnp.int32).reshape(NW, N_CHUNKS, CH)
    pos2d = pos_emb.reshape(seq, D)
    rows = _make_sc_stage(n_tok)(x_arr, token_table, pos2d, mod_table,
                                 gamma, beta)
    return _tc_ln_plain_half(rows.reshape(bsz, seq, D), pos2d,
                             mod_table[0:1], gamma.reshape(1, D),
                             beta.reshape(1, D), bsz, seq)


# speedup vs baseline: 1.5644x; 1.5644x over previous
import jax
import jax.numpy as jnp
from jax import lax
from jax.experimental import pallas as pl
from jax.experimental.pallas import tpu as pltpu
from jax.experimental.pallas import tpu_sc as plsc

D = 512
EPS = 1e-5
NW = 32
CHUNK = 64


def _make_sc_gather(n_tok):
    tok_per_w = n_tok // NW
    n_chunks = tok_per_w // CHUNK
    mesh = plsc.VectorSubcoreMesh(core_axis_name="c", subcore_axis_name="s")

    NBUF = 3

    def body(x_hbm, table_hbm, out_hbm, idx_v, b0, b1, b2, g0, g1, g2,
             o0, o1, o2):
        wid = lax.axis_index("s") * 2 + lax.axis_index("c")
        base = wid * tok_per_w
        pltpu.sync_copy(x_hbm.at[wid], idx_v)

        bufs = (b0, b1, b2)
        gsems = (g0, g1, g2)
        osems = (o0, o1, o2)

        def gather(c):
            return pltpu.async_copy(
                table_hbm.at[idx_v.at[c]], bufs[c % NBUF], gsems[c % NBUF])

        def put(c):
            return pltpu.async_copy(
                bufs[c % NBUF], out_hbm.at[pl.ds(base + c * CHUNK, CHUNK)],
                osems[c % NBUF])

        gathers = [None] * n_chunks
        puts = [None] * n_chunks
        for i in range(NBUF - 1):
            gathers[i] = gather(i)
        for c in range(n_chunks):
            nxt = c + NBUF - 1
            if nxt < n_chunks:
                if nxt - NBUF >= 0:
                    puts[nxt - NBUF].wait()
                gathers[nxt] = gather(nxt)
            gathers[c].wait()
            puts[c] = put(c)
        for c in range(n_chunks - NBUF, n_chunks):
            puts[c].wait()

    return pl.kernel(
        body,
        out_type=jax.ShapeDtypeStruct((n_tok, D), jnp.float32),
        mesh=mesh,
        scratch_types=[
            pltpu.VMEM((n_chunks, CHUNK), jnp.int32),
            pltpu.VMEM((CHUNK, D), jnp.float32),
            pltpu.VMEM((CHUNK, D), jnp.float32),
            pltpu.VMEM((CHUNK, D), jnp.float32),
            pltpu.SemaphoreType.DMA,
            pltpu.SemaphoreType.DMA,
            pltpu.SemaphoreType.DMA,
            pltpu.SemaphoreType.DMA,
            pltpu.SemaphoreType.DMA,
            pltpu.SemaphoreType.DMA,
        ],
    )


def _tc_ln_body(rows_ref, pos_ref, mod_ref, g_ref, b_ref, o_ref):
    h = rows_ref[...] + pos_ref[...][None] + mod_ref[...][None]
    mean = jnp.mean(h, axis=-1, keepdims=True)
    meansq = jnp.mean(h * h, axis=-1, keepdims=True)
    var = meansq - mean * mean
    scale = lax.rsqrt(var + EPS) * g_ref[...][None]
    shift = b_ref[...][None] - mean * scale
    o_ref[...] = h * scale + shift


def _tc_ln(rows3d, pos2d, mod_row, g2d, b2d, ts, bsz, seq):
    return pl.pallas_call(
        _tc_ln_body,
        grid=(seq // ts,),
        in_specs=[
            pl.BlockSpec((bsz, ts, D), lambda j: (0, j, 0)),
            pl.BlockSpec((ts, D), lambda j: (j, 0)),
            pl.BlockSpec((1, D), lambda j: (0, 0)),
            pl.BlockSpec((1, D), lambda j: (0, 0)),
            pl.BlockSpec((1, D), lambda j: (0, 0)),
        ],
        out_specs=pl.BlockSpec((bsz, ts, D), lambda j: (0, j, 0)),
        out_shape=jax.ShapeDtypeStruct((bsz, seq, D), jnp.float32),
    )(rows3d, pos2d, mod_row, g2d, b2d)


def kernel(x, token_table, pos_emb, mod_table, gamma, beta):
    bsz, seq = x.shape
    n_tok = bsz * seq
    n_chunks = n_tok // NW // CHUNK
    x_arr = x.astype(jnp.int32).reshape(NW, n_chunks, CHUNK)
    rows = _make_sc_gather(n_tok)(x_arr, token_table)
    pos2d = pos_emb.reshape(seq, D)
    return _tc_ln(rows.reshape(bsz, seq, D), pos2d, mod_table[0:1],
                  gamma.reshape(1, D), beta.reshape(1, D), 1024, bsz, seq)
